# baseline (speedup 1.0000x reference)
import functools

import jax
import jax.numpy as jnp
from jax import lax
from jax.experimental import pallas as pl
from jax.experimental.pallas import tpu as pltpu

NB = 6
H = 128
INT = 64
NBT = 5


def _silu(v):
    return v * jax.nn.sigmoid(v)


def _dot(a, b):
    return jnp.dot(a, b, preferred_element_type=jnp.float32)


def _pre_body(x_ref, rbf_ref, Wkj_ref, bkj_ref, Wr1_ref, Wr2_ref, Wd_ref,
              o_ref):
    x = x_ref[...]
    xkj = _silu(_dot(x, Wkj_ref[...]) + bkj_ref[...])
    r = _dot(_dot(rbf_ref[...], Wr1_ref[...]), Wr2_ref[...])
    o_ref[...] = _silu(_dot(xkj * r, Wd_ref[...]))


def _run_pre(x, rbf, W_kj, b_kj, W_rbf1, W_rbf2, W_down, block):
    n = x.shape[0]
    nrad = rbf.shape[1]
    bas = W_rbf1.shape[1]
    full = lambda shape: pl.BlockSpec(shape, lambda i: (0,) * len(shape))
    return pl.pallas_call(
        _pre_body,
        out_shape=jax.ShapeDtypeStruct((n, INT), jnp.float32),
        grid=(n // block,),
        in_specs=[
            pl.BlockSpec((block, H), lambda i: (i, 0)),
            pl.BlockSpec((block, nrad), lambda i: (i, 0)),
            full((H, H)),
            full((1, H)),
            full((nrad, bas)),
            full((bas, H)),
            full((H, INT)),
        ],
        out_specs=pl.BlockSpec((block, INT), lambda i: (i, 0)),
    )(x, rbf, W_kj, b_kj.reshape(1, H), W_rbf1, W_rbf2, W_down)


def _s2_body(sbf_ref, W1_ref, W2_ref, o_ref):
    o_ref[...] = _dot(_dot(sbf_ref[...], W1_ref[...]), W2_ref[...])


def _run_s2(sbf, W_sbf1, W_sbf2, block):
    t, nsr = sbf.shape
    bas = W_sbf1.shape[1]
    full = lambda shape: pl.BlockSpec(shape, lambda i: (0,) * len(shape))
    return pl.pallas_call(
        _s2_body,
        out_shape=jax.ShapeDtypeStruct((t, INT), jnp.float32),
        grid=(t // block,),
        in_specs=[
            pl.BlockSpec((block, nsr), lambda i: (i, 0)),
            full((nsr, bas)),
            full((bas, INT)),
        ],
        out_specs=pl.BlockSpec((block, INT), lambda i: (i, 0)),
    )(sbf, W_sbf1, W_sbf2)


def _post_body(x_ref, bkt_ref, a_ref, Wup_ref, Wji_ref, bji_ref, Wb1_ref,
               bb1_ref, Wb2_ref, bb2_ref, Wl_ref, bl_ref, Wa1_ref, ba1_ref,
               Wa2_ref, ba2_ref, o_ref):
    x = x_ref[...]
    bkt = bkt_ref[...]
    a = a_ref[0]
    Wup = Wup_ref[...]
    Wji = Wji_ref[...]
    bji = bji_ref[...]
    Wb1 = Wb1_ref[...]
    bb1 = bb1_ref[...]
    Wb2 = Wb2_ref[...]
    bb2 = bb2_ref[...]
    Wl = Wl_ref[...]
    bl = bl_ref[...]
    Wa1 = Wa1_ref[...]
    ba1 = ba1_ref[...]
    Wa2 = Wa2_ref[...]
    ba2 = ba2_ref[...]

    def branch(b, u):
        xjs = _silu(_dot(x, Wji[b]) + bji[b])
        if u is None:
            h = xjs
        else:
            h = xjs + _silu(_dot(u, Wup[b]))
        h = h + _silu(_dot(_silu(_dot(h, Wb1[b]) + bb1[b]), Wb2[b]) + bb2[b])
        h = _silu(_dot(h, Wl[b]) + bl[b]) + x
        h = h + _silu(_dot(_silu(_dot(h, Wa1[b]) + ba1[b]), Wa2[b]) + ba2[b])
        return h

    sub = [bkt[:, k * INT:(k + 1) * INT] for k in range(NBT)]
    gen = sub[0] + sub[1] + sub[2] + sub[3] + sub[4]
    acc = a * branch(NB - 1, gen)
    for b in range(NB):
        u = None if b == 0 else sub[b - 1]
        acc = acc + (1.0 - a) * branch(b, u)
    o_ref[...] = acc


def _run_post(x, buckets, alpha, W_up, W_ji, b_ji, Wb1, bb1, Wb2, bb2, W_lin,
              b_lin, Wa1, ba1, Wa2, ba2, block):
    n = x.shape[0]
    full = lambda shape: pl.BlockSpec(shape, lambda *_: (0,) * len(shape))
    wfull = lambda shape: pl.BlockSpec(shape, lambda *_: (0,) * len(shape))
    return pl.pallas_call(
        _post_body,
        out_shape=jax.ShapeDtypeStruct((n, H), jnp.float32),
        grid=(n // block,),
        in_specs=[
            pl.BlockSpec((block, H), lambda i: (i, 0)),
            pl.BlockSpec((block, NBT * INT), lambda i: (i, 0)),
            pl.BlockSpec(memory_space=pltpu.SMEM),
            wfull((NB, INT, H)),
            wfull((NB, H, H)),
            wfull((NB, H)),
            wfull((NB, H, H)),
            wfull((NB, H)),
            wfull((NB, H, H)),
            wfull((NB, H)),
            wfull((NB, H, H)),
            wfull((NB, H)),
            wfull((NB, H, H)),
            wfull((NB, H)),
            wfull((NB, H, H)),
            wfull((NB, H)),
        ],
        out_specs=pl.BlockSpec((block, H), lambda i: (i, 0)),
    )(x, buckets, alpha, W_up, W_ji, b_ji, Wb1, bb1, Wb2, bb2, W_lin, b_lin,
      Wa1, ba1, Wa2, ba2)


def _sparse_xla(xk, s2, idx_kj, idx_ji, bt, n):
    x_kj_t = xk[idx_kj] * s2
    cidx = idx_ji * NBT + bt[idx_kj]
    buckets = jax.ops.segment_sum(x_kj_t, cidx, num_segments=NBT * n)
    return buckets.reshape(n, NBT * INT)


def kernel(x, rbf, sbf, alpha, lambda_d, W_rbf1, W_rbf2, W_sbf1, W_sbf2,
           W_kj, b_kj, W_ji, b_ji, W_down, W_up, Wb1, bb1, Wb2, bb2, W_lin,
           b_lin, Wa1, ba1, Wa2, ba2, idx_kj, idx_ji, bt):
    n = x.shape[0]
    xk = _run_pre(x, rbf, W_kj, b_kj, W_rbf1, W_rbf2, W_down, block=1000)
    s2 = _run_s2(sbf, W_sbf1, W_sbf2, block=2000)
    buckets = _sparse_xla(xk, s2, idx_kj, idx_ji, bt, n)
    return _run_post(x, buckets, alpha, W_up, W_ji, b_ji, Wb1, bb1, Wb2, bb2,
                     W_lin, b_lin, Wa1, ba1, Wa2, ba2, block=500)


# speedup vs baseline: 1.5972x; 1.5972x over previous
import functools

import jax
import jax.numpy as jnp
from jax import lax
from jax.experimental import pallas as pl
from jax.experimental.pallas import tpu as pltpu
from jax.experimental.pallas import tpu_sc as plsc

NB = 6
H = 128
INT = 64
NBT = 5


def _silu(v):
    return v * jax.nn.sigmoid(v)


def _dot(a, b):
    return jnp.dot(a, b, preferred_element_type=jnp.float32)


def _pre_body(x_ref, rbf_ref, Wkj_ref, bkj_ref, Wr1_ref, Wr2_ref, Wd_ref,
              o_ref):
    x = x_ref[...]
    xkj = _silu(_dot(x, Wkj_ref[...]) + bkj_ref[...])
    r = _dot(_dot(rbf_ref[...], Wr1_ref[...]), Wr2_ref[...])
    o_ref[...] = _silu(_dot(xkj * r, Wd_ref[...]))


def _run_pre(x, rbf, W_kj, b_kj, W_rbf1, W_rbf2, W_down, block):
    n = x.shape[0]
    nrad = rbf.shape[1]
    bas = W_rbf1.shape[1]
    full = lambda shape: pl.BlockSpec(shape, lambda i: (0,) * len(shape))
    return pl.pallas_call(
        _pre_body,
        out_shape=jax.ShapeDtypeStruct((n, INT), jnp.float32),
        grid=(n // block,),
        in_specs=[
            pl.BlockSpec((block, H), lambda i: (i, 0)),
            pl.BlockSpec((block, nrad), lambda i: (i, 0)),
            full((H, H)),
            full((1, H)),
            full((nrad, bas)),
            full((bas, H)),
            full((H, INT)),
        ],
        out_specs=pl.BlockSpec((block, INT), lambda i: (i, 0)),
    )(x, rbf, W_kj, b_kj.reshape(1, H), W_rbf1, W_rbf2, W_down)


def _s2_body(sbf_ref, W1_ref, W2_ref, o_ref):
    o_ref[...] = _dot(_dot(sbf_ref[...], W1_ref[...]), W2_ref[...])


def _run_s2(sbf, W_sbf1, W_sbf2, block):
    t, nsr = sbf.shape
    bas = W_sbf1.shape[1]
    full = lambda shape: pl.BlockSpec(shape, lambda i: (0,) * len(shape))
    return pl.pallas_call(
        _s2_body,
        out_shape=jax.ShapeDtypeStruct((t, INT), jnp.float32),
        grid=(t // block,),
        in_specs=[
            pl.BlockSpec((block, nsr), lambda i: (i, 0)),
            full((nsr, bas)),
            full((bas, INT)),
        ],
        out_specs=pl.BlockSpec((block, INT), lambda i: (i, 0)),
    )(sbf, W_sbf1, W_sbf2)


def _post_body(x_ref, bkt_ref, a_ref, Wup_ref, Wji_ref, bji_ref, Wb1_ref,
               bb1_ref, Wb2_ref, bb2_ref, Wl_ref, bl_ref, Wa1_ref, ba1_ref,
               Wa2_ref, ba2_ref, o_ref):
    x = x_ref[...]
    bkt = bkt_ref[...]
    a = a_ref[0]
    Wup = Wup_ref[...]
    Wji = Wji_ref[...]
    bji = bji_ref[...]
    Wb1 = Wb1_ref[...]
    bb1 = bb1_ref[...]
    Wb2 = Wb2_ref[...]
    bb2 = bb2_ref[...]
    Wl = Wl_ref[...]
    bl = bl_ref[...]
    Wa1 = Wa1_ref[...]
    ba1 = ba1_ref[...]
    Wa2 = Wa2_ref[...]
    ba2 = ba2_ref[...]

    def branch(b, u):
        xjs = _silu(_dot(x, Wji[b]) + bji[b])
        if u is None:
            h = xjs
        else:
            h = xjs + _silu(_dot(u, Wup[b]))
        h = h + _silu(_dot(_silu(_dot(h, Wb1[b]) + bb1[b]), Wb2[b]) + bb2[b])
        h = _silu(_dot(h, Wl[b]) + bl[b]) + x
        h = h + _silu(_dot(_silu(_dot(h, Wa1[b]) + ba1[b]), Wa2[b]) + ba2[b])
        return h

    sub = [bkt[:, k * INT:(k + 1) * INT] for k in range(NBT)]
    gen = sub[0] + sub[1] + sub[2] + sub[3] + sub[4]
    acc = a * branch(NB - 1, gen)
    for b in range(NB):
        u = None if b == 0 else sub[b - 1]
        acc = acc + (1.0 - a) * branch(b, u)
    o_ref[...] = acc


def _run_post(x, buckets, alpha, W_up, W_ji, b_ji, Wb1, bb1, Wb2, bb2, W_lin,
              b_lin, Wa1, ba1, Wa2, ba2, block):
    n = x.shape[0]
    full = lambda shape: pl.BlockSpec(shape, lambda *_: (0,) * len(shape))
    wfull = lambda shape: pl.BlockSpec(shape, lambda *_: (0,) * len(shape))
    return pl.pallas_call(
        _post_body,
        out_shape=jax.ShapeDtypeStruct((n, H), jnp.float32),
        grid=(n // block,),
        in_specs=[
            pl.BlockSpec((block, H), lambda i: (i, 0)),
            pl.BlockSpec((block, NBT * INT), lambda i: (i, 0)),
            pl.BlockSpec(memory_space=pltpu.SMEM),
            wfull((NB, INT, H)),
            wfull((NB, H, H)),
            wfull((NB, H)),
            wfull((NB, H, H)),
            wfull((NB, H)),
            wfull((NB, H, H)),
            wfull((NB, H)),
            wfull((NB, H, H)),
            wfull((NB, H)),
            wfull((NB, H, H)),
            wfull((NB, H)),
            wfull((NB, H, H)),
            wfull((NB, H)),
        ],
        out_specs=pl.BlockSpec((block, H), lambda i: (i, 0)),
    )(x, buckets, alpha, W_up, W_ji, b_ji, Wb1, bb1, Wb2, bb2, W_lin, b_lin,
      Wa1, ba1, Wa2, ba2)


def _sparse_xla(xk, s2, idx_kj, idx_ji, bt, n):
    x_kj_t = xk[idx_kj] * s2
    cidx = idx_ji * NBT + bt[idx_kj]
    buckets = jax.ops.segment_sum(x_kj_t, cidx, num_segments=NBT * n)
    return buckets.reshape(n, NBT * INT)


_ROWS = 20480
_STRIPE = _ROWS // 16
_CB = 6000
_NV = _CB // 16
_K = 128
_IDCAP = 60032


def _sc_body(n, t, ts, nsub, passes_per_core,
             xk_ref, s2_ref, kj_ref, ji_ref, bt_ref, out_ref, cx2_ref,
             idsp_ref,
             ji_st, kj_st, btv_st, idsb, idb, cxidb, kjb, cxb, locb,
             xrw, s2w, zb, acc, sem_b, sem1, sem2, sem3):
    c = lax.axis_index("c")
    s = lax.axis_index("s")
    iota = lax.iota(jnp.int32, 16)
    zf = jnp.zeros((16,), jnp.float32)
    zi = jnp.zeros((16,), jnp.int32)
    tbase = (c * 16 + s) * _IDCAP

    def _z0(r, _):
        for q in range(4):
            zb[r, pl.ds(q * 16, 16)] = zf
        return 0
    lax.fori_loop(0, zb.shape[0], _z0, 0)

    def _z1(i, _):
        idsb[pl.ds(i * 16, 16)] = zi
        return 0
    lax.fori_loop(0, idsb.shape[0] // 16, _z1, 0)

    def _build(j, _):
        base = s * ts + j * _CB
        pltpu.sync_copy(ji_ref.at[pl.ds(base, _CB)], ji_st)
        pltpu.sync_copy(kj_ref.at[pl.ds(base, _CB)], kj_st)
        descs = []
        for g in range(_CB // 120):
            descs.append(pltpu.async_copy(
                bt_ref.at[kj_st.at[pl.ds(g * 120, 120)]],
                btv_st.at[pl.ds(g * 120, 120)], sem_b))
        for d in descs:
            d.wait()

        def _cw(v, _):
            jiv = ji_st[pl.ds(v * 16, 16)]
            btv = btv_st[pl.ds(v * 16, 16)]
            ji_st[pl.ds(v * 16, 16)] = jiv * NBT + btv
            return 0
        lax.fori_loop(0, _NV, _cw, 0)
        pltpu.sync_copy(ji_st, cx2_ref.at[pl.ds(c * t + base, _CB)])
        return 0
    lax.fori_loop(0, nsub, _build, 0)
    plsc.subcore_barrier()

    def _pass(g, _):
        p = passes_per_core * c + g
        lo = p * _ROWS
        for r in range(_STRIPE // 128):
            pltpu.sync_copy(zb, acc.at[pl.ds(s * _STRIPE + r * 128, 128)])
        plsc.subcore_barrier()

        def _fj(j, cnt):
            base = s * ts + j * _CB
            pltpu.sync_copy(cx2_ref.at[pl.ds(c * t + base, _CB)], kj_st)

            def _fv(v, cnt):
                cid = kj_st[pl.ds(v * 16, 16)]
                m = (cid >= lo) & (cid < lo + _ROWS)
                mi = jnp.where(m, 1, 0)
                idv = base + v * 16 + iota
                _, sv = plsc.sort_key_val(mi, idv, descending=True)
                npop = plsc.all_reduce_population_count(m)
                plsc.store_scatter(idsb, [(cnt + iota) & 1023], sv,
                                   mask=iota < npop)
                cnt2 = cnt + npop[0]

                @pl.when((cnt2 >> 7) != (cnt >> 7))
                def _flush():
                    blk = cnt >> 7
                    pltpu.sync_copy(
                        idsb.at[pl.ds((blk & 7) * 128, 128)],
                        idsp_ref.at[pl.ds(tbase + blk * 128, 128)])
                return cnt2
            return lax.fori_loop(0, _NV, _fv, cnt)
        cnt = lax.fori_loop(0, nsub, _fj, jnp.int32(0))
        blk = cnt >> 7
        pltpu.sync_copy(idsb.at[pl.ds((blk & 7) * 128, 128)],
                        idsp_ref.at[pl.ds(tbase + blk * 128, 128)])

        nch = (cnt + _K - 1) >> 7

        def _pb(b2, _):
            pltpu.sync_copy(idsp_ref.at[pl.ds(tbase + b2 * _K, _K)], idb)
            for v8 in range(_K // 16):
                idv = idb[pl.ds(v8 * 16, 16)]
                cxidb[pl.ds(v8 * 16, 16)] = idv + c * t
            d1 = pltpu.async_copy(s2_ref.at[idb], s2w, sem1)
            d2 = pltpu.async_copy(kj_ref.at[idb], kjb, sem2)
            d3 = pltpu.async_copy(cx2_ref.at[cxidb], cxb, sem3)
            d2.wait()
            d4 = pltpu.async_copy(xk_ref.at[kjb], xrw, sem2)
            d3.wait()
            for v8 in range(_K // 16):
                cx = cxb[pl.ds(v8 * 16, 16)]
                pos = b2 * _K + v8 * 16 + iota
                locb[pl.ds(v8 * 16, 16)] = jnp.where(
                    pos < cnt, cx - lo, _ROWS + s)
            d1.wait()
            d4.wait()

            def _mul(r, _):
                for q in range(4):
                    xrw[r, pl.ds(q * 16, 16)] = (
                        xrw[r, pl.ds(q * 16, 16)] * s2w[r, pl.ds(q * 16, 16)])
                return 0
            lax.fori_loop(0, _K, _mul, 0)
            pltpu.sync_copy(xrw, acc.at[locb], add=True)
            return 0
        lax.fori_loop(0, nch, _pb, 0)
        plsc.subcore_barrier()

        for r in range(_STRIPE // 128):
            row = s * _STRIPE + r * 128
            pltpu.sync_copy(acc.at[pl.ds(row, 128)],
                            out_ref.at[pl.ds(lo + row, 128)])
        return 0
    lax.fori_loop(0, passes_per_core, _pass, 0)


def _sparse_sc(xk, s2, idx_kj, idx_ji, bt):
    n = xk.shape[0]
    t = s2.shape[0]
    passes = -(-(n * NBT) // _ROWS)
    if passes % 2:
        passes += 1
    ppc = passes // 2
    ts = t // 16
    nsub = ts // _CB
    mesh = plsc.VectorSubcoreMesh(core_axis_name="c", subcore_axis_name="s")
    body = functools.partial(_sc_body, n, t, ts, nsub, ppc)
    out, _, _ = pl.kernel(
        body,
        out_type=(
            jax.ShapeDtypeStruct((passes * _ROWS, INT), jnp.float32),
            jax.ShapeDtypeStruct((2 * t,), jnp.int32),
            jax.ShapeDtypeStruct((32 * _IDCAP,), jnp.int32),
        ),
        mesh=mesh,
        compiler_params=pltpu.CompilerParams(needs_layout_passes=False, use_tc_tiling_on_sc=False),
        scratch_types=[
            pltpu.VMEM((_CB,), jnp.int32),
            pltpu.VMEM((_CB,), jnp.int32),
            pltpu.VMEM((_CB,), jnp.int32),
            pltpu.VMEM((1024,), jnp.int32),
            pltpu.VMEM((_K,), jnp.int32),
            pltpu.VMEM((_K,), jnp.int32),
            pltpu.VMEM((_K,), jnp.int32),
            pltpu.VMEM((_K,), jnp.int32),
            pltpu.VMEM((_K,), jnp.int32),
            pltpu.VMEM((_K, INT), jnp.float32),
            pltpu.VMEM((_K, INT), jnp.float32),
            pltpu.VMEM((_K, INT), jnp.float32),
            pltpu.VMEM_SHARED((_ROWS + 16, INT), jnp.float32),
            pltpu.SemaphoreType.DMA,
            pltpu.SemaphoreType.DMA,
            pltpu.SemaphoreType.DMA,
            pltpu.SemaphoreType.DMA,
        ],
    )(xk, s2, idx_kj, idx_ji, bt)
    return out[:n * NBT].reshape(n, NBT * INT)


def kernel(x, rbf, sbf, alpha, lambda_d, W_rbf1, W_rbf2, W_sbf1, W_sbf2,
           W_kj, b_kj, W_ji, b_ji, W_down, W_up, Wb1, bb1, Wb2, bb2, W_lin,
           b_lin, Wa1, ba1, Wa2, ba2, idx_kj, idx_ji, bt):
    n = x.shape[0]
    xk = _run_pre(x, rbf, W_kj, b_kj, W_rbf1, W_rbf2, W_down, block=1000)
    s2 = _run_s2(sbf, W_sbf1, W_sbf2, block=2000)
    buckets = _sparse_sc(xk, s2, idx_kj, idx_ji, bt)
    return _run_post(x, buckets, alpha, W_up, W_ji, b_ji, Wb1, bb1, Wb2, bb2,
                     W_lin, b_lin, Wa1, ba1, Wa2, ba2, block=512)
